# baseline (speedup 1.0000x reference)
import jax
import jax.numpy as jnp
from jax.experimental import pallas as pl
from jax.experimental.pallas import tpu as pltpu

_ROWS = 16384
_COLS = 2048
_BLOCK_ROWS = 1024
_GRID = _ROWS // _BLOCK_ROWS


def _mse_body(p_ref, t_ref, m_ref, sq_ref, cnt_ref):
    i = pl.program_id(0)
    m = m_ref[...]
    d = jnp.where(m, p_ref[...] - t_ref[...], 0.0)
    sq = jnp.sum(d * d)
    c = jnp.sum(m.astype(jnp.float32))

    @pl.when(i == 0)
    def _init():
        sq_ref[0, 0] = sq
        cnt_ref[0, 0] = c

    @pl.when(i > 0)
    def _acc():
        sq_ref[0, 0] += sq
        cnt_ref[0, 0] += c


def kernel(y_pred, y_true, mask):
    p = y_pred.reshape(_ROWS, _COLS)
    t = y_true.reshape(_ROWS, _COLS)
    m = mask.reshape(_ROWS, _COLS)
    in_spec = pl.BlockSpec((_BLOCK_ROWS, _COLS), lambda i: (i, 0))
    out_spec = pl.BlockSpec((1, 1), lambda i: (0, 0), memory_space=pltpu.SMEM)
    sq, cnt = pl.pallas_call(
        _mse_body,
        grid=(_GRID,),
        in_specs=[in_spec, in_spec, in_spec],
        out_specs=[out_spec, out_spec],
        out_shape=[
            jax.ShapeDtypeStruct((1, 1), jnp.float32),
            jax.ShapeDtypeStruct((1, 1), jnp.float32),
        ],
    )(p, t, m)
    return sq[0, 0] / cnt[0, 0]


# speedup vs baseline: 1.0092x; 1.0092x over previous
import jax
import jax.numpy as jnp
from jax.experimental import pallas as pl
from jax.experimental.pallas import tpu as pltpu

_ROWS = 16384
_COLS = 2048
_BLOCK_ROWS = 512
_GRID = _ROWS // _BLOCK_ROWS


def _mse_body(p_ref, t_ref, m_ref, sq_ref, cnt_ref):
    i = pl.program_id(0)
    m = m_ref[...]
    d = jnp.where(m, p_ref[...] - t_ref[...], 0.0)
    sq = jnp.sum(d * d)
    c = jnp.sum(m.astype(jnp.float32))

    @pl.when(i == 0)
    def _init():
        sq_ref[0, 0] = sq
        cnt_ref[0, 0] = c

    @pl.when(i > 0)
    def _acc():
        sq_ref[0, 0] += sq
        cnt_ref[0, 0] += c


def kernel(y_pred, y_true, mask):
    p = y_pred.reshape(_ROWS, _COLS)
    t = y_true.reshape(_ROWS, _COLS)
    m = mask.reshape(_ROWS, _COLS)
    in_spec = pl.BlockSpec((_BLOCK_ROWS, _COLS), lambda i: (i, 0))
    out_spec = pl.BlockSpec((1, 1), lambda i: (0, 0), memory_space=pltpu.SMEM)
    sq, cnt = pl.pallas_call(
        _mse_body,
        grid=(_GRID,),
        in_specs=[in_spec, in_spec, in_spec],
        out_specs=[out_spec, out_spec],
        out_shape=[
            jax.ShapeDtypeStruct((1, 1), jnp.float32),
            jax.ShapeDtypeStruct((1, 1), jnp.float32),
        ],
    )(p, t, m)
    return sq[0, 0] / cnt[0, 0]


# speedup vs baseline: 1.1046x; 1.0945x over previous
import jax
import jax.numpy as jnp
from jax.experimental import pallas as pl
from jax.experimental.pallas import tpu as pltpu

_ROWS = 16384
_COLS = 2048
_BLOCK_ROWS = 512
_GRID = _ROWS // _BLOCK_ROWS
_BRW = _BLOCK_ROWS // 4


def _mse_body(p_ref, t_ref, m_hbm, sq_ref, cnt_ref, w_scr, sems):
    i = pl.program_id(0)
    wv = m_hbm.bitcast(jnp.uint32)

    def cp(j, slot):
        return pltpu.make_async_copy(
            wv.at[pl.ds(j * _BRW, _BRW), :], w_scr.at[slot], sems.at[slot]
        )

    @pl.when(i == 0)
    def _first():
        cp(0, 0).start()

    @pl.when(i + 1 < _GRID)
    def _prefetch():
        cp(i + 1, (i + 1) % 2).start()

    cp(i, i % 2).wait()
    w = w_scr[i % 2]

    shifts = jax.lax.broadcasted_iota(jnp.uint32, (1, 4, 1), 1) * 8
    bits = (w[:, None, :] >> shifts) & jnp.uint32(1)
    msk = bits.reshape(_BLOCK_ROWS, _COLS) != 0

    d = p_ref[...] - t_ref[...]
    dm = jnp.where(msk, d, 0.0)
    sq = jnp.sum(dm * dm)
    c_words = (w * jnp.uint32(0x01010101)) >> 24
    c = jnp.sum(c_words.astype(jnp.float32))

    @pl.when(i == 0)
    def _init():
        sq_ref[0, 0] = sq
        cnt_ref[0, 0] = c

    @pl.when(i > 0)
    def _acc():
        sq_ref[0, 0] += sq
        cnt_ref[0, 0] += c


def kernel(y_pred, y_true, mask):
    p = y_pred.reshape(_ROWS, _COLS)
    t = y_true.reshape(_ROWS, _COLS)
    m = mask.reshape(_ROWS, _COLS).astype(jnp.uint8)
    data_spec = pl.BlockSpec((_BLOCK_ROWS, _COLS), lambda i: (i, 0))
    out_spec = pl.BlockSpec((1, 1), lambda i: (0, 0), memory_space=pltpu.SMEM)
    sq, cnt = pl.pallas_call(
        _mse_body,
        grid=(_GRID,),
        in_specs=[data_spec, data_spec, pl.BlockSpec(memory_space=pl.ANY)],
        out_specs=[out_spec, out_spec],
        out_shape=[
            jax.ShapeDtypeStruct((1, 1), jnp.float32),
            jax.ShapeDtypeStruct((1, 1), jnp.float32),
        ],
        scratch_shapes=[
            pltpu.VMEM((2, _BRW, _COLS), jnp.uint32),
            pltpu.SemaphoreType.DMA((2,)),
        ],
    )(p, t, m)
    return sq[0, 0] / cnt[0, 0]


# speedup vs baseline: 1.1051x; 1.0005x over previous
import jax
import jax.numpy as jnp
from jax.experimental import pallas as pl
from jax.experimental.pallas import tpu as pltpu

_ROWS = 16384
_COLS = 2048
_BLOCK_ROWS = 512
_GRID = _ROWS // _BLOCK_ROWS
_BRW = _BLOCK_ROWS // 4
_DEPTH = 4


def _mse_body(p_ref, t_ref, m_hbm, sq_ref, cnt_ref, w_scr, sems):
    i = pl.program_id(0)
    wv = m_hbm.bitcast(jnp.uint32)

    def cp(j, slot):
        return pltpu.make_async_copy(
            wv.at[pl.ds(j * _BRW, _BRW), :], w_scr.at[slot], sems.at[slot]
        )

    @pl.when(i == 0)
    def _warmup():
        for j in range(_DEPTH - 1):
            cp(j, j).start()

    @pl.when(i + _DEPTH - 1 < _GRID)
    def _prefetch():
        j = i + _DEPTH - 1
        cp(j, j % _DEPTH).start()

    cp(i, i % _DEPTH).wait()
    w = w_scr[i % _DEPTH]

    shifts = jax.lax.broadcasted_iota(jnp.uint32, (1, 4, 1), 1) * 8
    bits = (w[:, None, :] >> shifts) & jnp.uint32(1)
    msk = bits.reshape(_BLOCK_ROWS, _COLS) != 0

    d = p_ref[...] - t_ref[...]
    dm = jnp.where(msk, d, 0.0)
    sq = jnp.sum(dm * dm)
    c_words = (w * jnp.uint32(0x01010101)) >> 24
    c = jnp.sum(c_words.astype(jnp.float32))

    @pl.when(i == 0)
    def _init():
        sq_ref[0, 0] = sq
        cnt_ref[0, 0] = c

    @pl.when(i > 0)
    def _acc():
        sq_ref[0, 0] += sq
        cnt_ref[0, 0] += c


def kernel(y_pred, y_true, mask):
    p = y_pred.reshape(_ROWS, _COLS)
    t = y_true.reshape(_ROWS, _COLS)
    m = mask.reshape(_ROWS, _COLS).astype(jnp.uint8)
    data_spec = pl.BlockSpec((_BLOCK_ROWS, _COLS), lambda i: (i, 0))
    out_spec = pl.BlockSpec((1, 1), lambda i: (0, 0), memory_space=pltpu.SMEM)
    sq, cnt = pl.pallas_call(
        _mse_body,
        grid=(_GRID,),
        in_specs=[data_spec, data_spec, pl.BlockSpec(memory_space=pl.ANY)],
        out_specs=[out_spec, out_spec],
        out_shape=[
            jax.ShapeDtypeStruct((1, 1), jnp.float32),
            jax.ShapeDtypeStruct((1, 1), jnp.float32),
        ],
        scratch_shapes=[
            pltpu.VMEM((_DEPTH, _BRW, _COLS), jnp.uint32),
            pltpu.SemaphoreType.DMA((_DEPTH,)),
        ],
    )(p, t, m)
    return sq[0, 0] / cnt[0, 0]


# speedup vs baseline: 1.4793x; 1.3386x over previous
import jax
import jax.numpy as jnp
from jax.experimental import pallas as pl
from jax.experimental.pallas import tpu as pltpu

_ROWS = 16384
_COLS = 2048
_BLOCK_ROWS = 512
_GRID = _ROWS // _BLOCK_ROWS
_BRW = _BLOCK_ROWS // 4
_DEPTH = 4


def _mse_body(p_ref, t_ref, m_hbm, sq_ref, cnt_ref, w_scr, sems):
    i = pl.program_id(0)
    wv = m_hbm.bitcast(jnp.uint32)

    def cp(j, slot):
        return pltpu.make_async_copy(
            wv.at[pl.ds(j * _BRW, _BRW), :], w_scr.at[slot], sems.at[slot]
        )

    @pl.when(i == 0)
    def _warmup():
        for j in range(_DEPTH - 1):
            cp(j, j).start()

    @pl.when(i + _DEPTH - 1 < _GRID)
    def _prefetch():
        j = i + _DEPTH - 1
        cp(j, j % _DEPTH).start()

    cp(i, i % _DEPTH).wait()
    w = w_scr[i % _DEPTH]

    msk = pltpu.bitcast(w, jnp.uint8) != 0

    d = p_ref[...] - t_ref[...]
    dm = jnp.where(msk, d, 0.0)
    sq = jnp.sum(dm * dm)
    c_words = (w * jnp.uint32(0x01010101)) >> 24
    c = jnp.sum(c_words.astype(jnp.float32))

    @pl.when(i == 0)
    def _init():
        sq_ref[0, 0] = sq
        cnt_ref[0, 0] = c

    @pl.when(i > 0)
    def _acc():
        sq_ref[0, 0] += sq
        cnt_ref[0, 0] += c


def kernel(y_pred, y_true, mask):
    p = y_pred.reshape(_ROWS, _COLS)
    t = y_true.reshape(_ROWS, _COLS)
    m = mask.reshape(_ROWS, _COLS).astype(jnp.uint8)
    data_spec = pl.BlockSpec((_BLOCK_ROWS, _COLS), lambda i: (i, 0))
    out_spec = pl.BlockSpec((1, 1), lambda i: (0, 0), memory_space=pltpu.SMEM)
    sq, cnt = pl.pallas_call(
        _mse_body,
        grid=(_GRID,),
        in_specs=[data_spec, data_spec, pl.BlockSpec(memory_space=pl.ANY)],
        out_specs=[out_spec, out_spec],
        out_shape=[
            jax.ShapeDtypeStruct((1, 1), jnp.float32),
            jax.ShapeDtypeStruct((1, 1), jnp.float32),
        ],
        scratch_shapes=[
            pltpu.VMEM((_DEPTH, _BRW, _COLS), jnp.uint32),
            pltpu.SemaphoreType.DMA((_DEPTH,)),
        ],
    )(p, t, m)
    return sq[0, 0] / cnt[0, 0]
